# Initial kernel scaffold; baseline (speedup 1.0000x reference)
import functools

import jax
import jax.numpy as jnp
from jax import lax
from jax.experimental import pallas as pl
from jax.experimental.pallas import tpu as pltpu
from jax.experimental.pallas import tpu_sc as plsc

N = 10000
E = 320000
H = 128
NC = 2
NS = 16
NW = NC * NS
EPW = E // NW
CH = 80
NIT = EPW // CH
RPT = N // NS

_PREC = lax.Precision.HIGHEST


def _sc_scatter_body(x_hbm, src_hbm, dst_hbm, zero_hbm, out_hbm,
                     sidx, didx, rows, acc, sem):
    c = lax.axis_index("c")
    s = lax.axis_index("s")
    wid = s * NC + c
    r0 = s * RPT

    @pl.when(c == 0)
    def _():
        pltpu.sync_copy(x_hbm.at[pl.ds(r0, RPT)], acc.at[pl.ds(r0, RPT)])

    @pl.when(c != 0)
    def _():
        pltpu.sync_copy(zero_hbm.at[pl.ds(r0, RPT)], acc.at[pl.ds(r0, RPT)])

    plsc.subcore_barrier()

    base = wid * EPW

    def body(i, carry):
        off = base + i * CH
        pltpu.sync_copy(src_hbm.at[pl.ds(off, CH)], sidx)
        pltpu.sync_copy(dst_hbm.at[pl.ds(off, CH)], didx)
        pltpu.async_copy(x_hbm.at[sidx], rows, sem).wait()
        pltpu.sync_copy(rows, acc.at[didx], add=True)
        return carry

    lax.fori_loop(0, NIT, body, 0)

    plsc.subcore_barrier()
    pltpu.sync_copy(acc.at[pl.ds(r0, RPT)], out_hbm.at[c, pl.ds(r0, RPT)])


def _make_sc_scatter():
    mesh = plsc.VectorSubcoreMesh(
        core_axis_name="c", subcore_axis_name="s",
        num_cores=NC, num_subcores=NS)
    return functools.partial(
        pl.kernel,
        out_type=jax.ShapeDtypeStruct((NC, N, H), jnp.float32),
        mesh=mesh,
        scratch_types=[
            pltpu.VMEM((CH,), jnp.int32),
            pltpu.VMEM((CH,), jnp.int32),
            pltpu.VMEM((CH, H), jnp.float32),
            pltpu.VMEM_SHARED((N, H), jnp.float32),
            pltpu.SemaphoreType.DMA,
        ],
    )(_sc_scatter_body)


_BLK = 1000
_NBLK = N // _BLK


def _dot_t(a, w):
    return lax.dot_general(a, w, (((1,), (1,)), ((), ())),
                           preferred_element_type=jnp.float32,
                           precision=_PREC)


def _tc_body(p0, p1, w_d1, b_d1, w_d2, b_d2, tfv, w_t1, b_t1, w_t2, b_t2,
             w_p1a, w_p1b, b_p1, w_p2, b_p2, out, acc):
    i = pl.program_id(0)

    @pl.when(i == 0)
    def _():
        acc[...] = jnp.zeros_like(acc)

    pre = p0[...] + p1[...]
    t = jax.nn.relu(_dot_t(pre, w_d1[...]) + b_d1[...])
    u = jax.nn.relu(_dot_t(t, w_d2[...]) + b_d2[...])
    acc[...] += jnp.sum(u.reshape(_BLK // 8, 8, H), axis=0)

    @pl.when(i == _NBLK - 1)
    def _():
        drug = jnp.sum(acc[...], axis=0, keepdims=True) * (1.0 / N)
        th = jax.nn.relu(_dot_t(tfv[...], w_t1[...]) + b_t1[...])
        temb = _dot_t(th, w_t2[...]) + b_t2[...]
        z = jax.nn.relu(_dot_t(drug, w_p1a[...]) + _dot_t(temb, w_p1b[...])
                        + b_p1[...])
        p = _dot_t(z, w_p2[...]) + b_p2[...]
        out[...] = jax.nn.sigmoid(p)


def _make_tc():
    row_spec = pl.BlockSpec((_BLK, H), lambda i: (i, 0))
    full = lambda shape: pl.BlockSpec(shape, lambda i: (0,) * len(shape))
    return pl.pallas_call(
        _tc_body,
        grid=(_NBLK,),
        in_specs=[
            row_spec, row_spec,
            full((H, H)), full((1, H)),
            full((H, H)), full((1, H)),
            full((1, H)),
            full((H, H)), full((1, H)),
            full((H, H)), full((1, H)),
            full((H, H)), full((H, H)), full((1, H)),
            full((1, H)), full((1, 1)),
        ],
        out_specs=pl.BlockSpec((1, 1), lambda i: (0, 0)),
        out_shape=jax.ShapeDtypeStruct((1, 1), jnp.float32),
        scratch_shapes=[pltpu.VMEM((8, H), jnp.float32)],
    )


def kernel(x, edge_index, target_feat_vec, W_d1, b_d1, W_d2, b_d2,
           W_t1, b_t1, W_t2, b_t2, W_p1, b_p1, W_p2, b_p2):
    src = edge_index[0]
    dst = edge_index[1]
    zeros = jnp.zeros((N, H), dtype=jnp.float32)

    partials = _make_sc_scatter()(x, src, dst, zeros)

    out = _make_tc()(
        partials[0], partials[1],
        W_d1, b_d1.reshape(1, H),
        W_d2, b_d2.reshape(1, H),
        target_feat_vec.reshape(1, H),
        W_t1, b_t1.reshape(1, H),
        W_t2, b_t2.reshape(1, H),
        W_p1[:, :H], W_p1[:, H:], b_p1.reshape(1, H),
        W_p2, b_p2.reshape(1, 1),
    )
    return out


# speedup vs baseline: 7.0832x; 7.0832x over previous
import functools

import jax
import jax.numpy as jnp
from jax import lax
from jax.experimental import pallas as pl
from jax.experimental.pallas import tpu as pltpu
from jax.experimental.pallas import tpu_sc as plsc

N = 10000
E = 320000
H = 128
NC = 2
NS = 16
NW = NC * NS
EPW = E // NW
CH = 80
NIT = EPW // CH
RPT = 624
RTAIL = N - NS * RPT

_PREC = lax.Precision.HIGHEST


def _sc_scatter_body(x_hbm, src_hbm, dst_hbm, zero_hbm, out_hbm,
                     sidx, didx, rows, acc, sem):
    c = lax.axis_index("c")
    s = lax.axis_index("s")
    wid = s * NC + c
    r0 = s * RPT

    @pl.when(c == 0)
    def _():
        pltpu.sync_copy(x_hbm.at[pl.ds(r0, RPT)], acc.at[pl.ds(r0, RPT)])

        @pl.when(s == NS - 1)
        def _():
            pltpu.sync_copy(x_hbm.at[pl.ds(NS * RPT, RTAIL)],
                            acc.at[pl.ds(NS * RPT, RTAIL)])

    @pl.when(c != 0)
    def _():
        pltpu.sync_copy(zero_hbm.at[pl.ds(r0, RPT)], acc.at[pl.ds(r0, RPT)])

        @pl.when(s == NS - 1)
        def _():
            pltpu.sync_copy(zero_hbm.at[pl.ds(NS * RPT, RTAIL)],
                            acc.at[pl.ds(NS * RPT, RTAIL)])

    plsc.subcore_barrier()

    base = wid * EPW

    def body(i, carry):
        off = base + i * CH
        pltpu.sync_copy(src_hbm.at[pl.ds(off, CH)], sidx)
        pltpu.sync_copy(dst_hbm.at[pl.ds(off, CH)], didx)
        pltpu.async_copy(x_hbm.at[sidx], rows, sem).wait()
        pltpu.sync_copy(rows, acc.at[didx], add=True)
        return carry

    lax.fori_loop(0, NIT, body, 0)

    plsc.subcore_barrier()
    pltpu.sync_copy(acc.at[pl.ds(r0, RPT)], out_hbm.at[c, pl.ds(r0, RPT)])

    @pl.when(s == NS - 1)
    def _():
        pltpu.sync_copy(acc.at[pl.ds(NS * RPT, RTAIL)],
                        out_hbm.at[c, pl.ds(NS * RPT, RTAIL)])


def _make_sc_scatter():
    mesh = plsc.VectorSubcoreMesh(
        core_axis_name="c", subcore_axis_name="s",
        num_cores=NC, num_subcores=NS)
    return functools.partial(
        pl.kernel,
        out_type=jax.ShapeDtypeStruct((NC, N, H), jnp.float32),
        mesh=mesh,
        scratch_types=[
            pltpu.VMEM((CH,), jnp.int32),
            pltpu.VMEM((CH,), jnp.int32),
            pltpu.VMEM((CH, H), jnp.float32),
            pltpu.VMEM_SHARED((N, H), jnp.float32),
            pltpu.SemaphoreType.DMA,
        ],
    )(_sc_scatter_body)


_BLK = 1000
_NBLK = N // _BLK


def _dot_t(a, w):
    return lax.dot_general(a, w, (((1,), (1,)), ((), ())),
                           preferred_element_type=jnp.float32,
                           precision=_PREC)


def _tc_body(p0, p1, w_d1, b_d1, w_d2, b_d2, tfv, w_t1, b_t1, w_t2, b_t2,
             w_p1a, w_p1b, b_p1, w_p2, b_p2, out, acc):
    i = pl.program_id(0)

    @pl.when(i == 0)
    def _():
        acc[...] = jnp.zeros_like(acc)

    pre = p0[...] + p1[...]
    t = jax.nn.relu(_dot_t(pre, w_d1[...]) + b_d1[...])
    u = jax.nn.relu(_dot_t(t, w_d2[...]) + b_d2[...])
    acc[...] += jnp.sum(u.reshape(_BLK // 8, 8, H), axis=0)

    @pl.when(i == _NBLK - 1)
    def _():
        drug = jnp.sum(acc[...], axis=0, keepdims=True) * (1.0 / N)
        th = jax.nn.relu(_dot_t(tfv[...], w_t1[...]) + b_t1[...])
        temb = _dot_t(th, w_t2[...]) + b_t2[...]
        z = jax.nn.relu(_dot_t(drug, w_p1a[...]) + _dot_t(temb, w_p1b[...])
                        + b_p1[...])
        p = jnp.sum(z * w_p2[...]) + b_p2[...]
        out[...] = jax.nn.sigmoid(p)


def _make_tc():
    row_spec = pl.BlockSpec((_BLK, H), lambda i: (i, 0))
    full = lambda shape: pl.BlockSpec(shape, lambda i: (0,) * len(shape))
    return pl.pallas_call(
        _tc_body,
        grid=(_NBLK,),
        in_specs=[
            row_spec, row_spec,
            full((H, H)), full((1, H)),
            full((H, H)), full((1, H)),
            full((1, H)),
            full((H, H)), full((1, H)),
            full((H, H)), full((1, H)),
            full((H, H)), full((H, H)), full((1, H)),
            full((1, H)), full((1, 1)),
        ],
        out_specs=pl.BlockSpec((1, 1), lambda i: (0, 0)),
        out_shape=jax.ShapeDtypeStruct((1, 1), jnp.float32),
        scratch_shapes=[pltpu.VMEM((8, H), jnp.float32)],
    )


def kernel(x, edge_index, target_feat_vec, W_d1, b_d1, W_d2, b_d2,
           W_t1, b_t1, W_t2, b_t2, W_p1, b_p1, W_p2, b_p2):
    src = edge_index[0]
    dst = edge_index[1]
    zeros = jnp.zeros((N, H), dtype=jnp.float32)

    partials = _make_sc_scatter()(x, src, dst, zeros)

    out = _make_tc()(
        partials[0], partials[1],
        W_d1, b_d1.reshape(1, H),
        W_d2, b_d2.reshape(1, H),
        target_feat_vec.reshape(1, H),
        W_t1, b_t1.reshape(1, H),
        W_t2, b_t2.reshape(1, H),
        W_p1[:, :H], W_p1[:, H:], b_p1.reshape(1, H),
        W_p2, b_p2.reshape(1, 1),
    )
    return out
